# Initial kernel scaffold; baseline (speedup 1.0000x reference)
import jax
import jax.numpy as jnp
from jax.experimental import pallas as pl


def kernel(x, h, embed, Wr1, br1, Wr2, br2, Wm, bm, Wv, W_out, b_out):
    raise NotImplementedError("write your pallas kernel here")


# speedup vs baseline: 22.8824x; 22.8824x over previous
import jax
import jax.numpy as jnp
from jax.experimental import pallas as pl
from jax.experimental.pallas import tpu as pltpu

_N = 512
_D = 64
_L = 2
_BI = 32
_NBLK = _N // _BI


def _silu(z):
    return z * jax.nn.sigmoid(z)


def _mace_body(pos_col, pos_row, embed, wr1, br1, wr2, br2, wm, bm, wv,
               wout, bout, vec_ref, inv_ref, feats_scr):
    l = pl.program_id(0)
    ib = pl.program_id(1)
    i0 = ib * _BI

    @pl.when(jnp.logical_and(l == 0, ib == 0))
    def _init():
        feats_scr[0] = jnp.broadcast_to(embed[:], (_N, _D))

    pc = pos_col[pl.ds(i0, _BI), :]
    pr = pos_row[:]
    d0 = pc[:, 0:1] - pr[0:1, :]
    d1 = pc[:, 1:2] - pr[1:2, :]
    d2 = pc[:, 2:3] - pr[2:3, :]
    r2 = d0 * d0 + d1 * d1 + d2 * d2 + 1e-6
    rlen = jnp.sqrt(r2)
    inv_r = 1.0 / rlen

    feats_all = feats_scr[pl.ds(l, 1), :, :][0]

    w1 = wr1[0]
    b1 = br1[0]
    pre = rlen[:, :, None] * w1[None, :, :] + b1[None, :, :]
    hid = _silu(pre)
    radial = jax.lax.dot_general(
        hid, wr2[0], (((2,), (0,)), ((), ())),
        preferred_element_type=jnp.float32) + br2[0][None, :, :]

    jj = jax.lax.broadcasted_iota(jnp.int32, (_BI, _N, _D), 1)
    ii = jax.lax.broadcasted_iota(jnp.int32, (_BI, _N, _D), 0) + i0
    radial = jnp.where(jj == ii, 0.0, radial)

    m = radial * feats_all[None, :, :]
    agg = jnp.sum(m, axis=1) * (1.0 / _N)

    f_old = feats_scr[pl.ds(l, 1), pl.ds(i0, _BI), :][0]
    z = jax.lax.dot_general(
        agg, wm[0], (((1,), (0,)), ((), ())),
        preferred_element_type=jnp.float32) + bm[0]
    f_new = f_old + _silu(z)

    @pl.when(l == 0)
    def _store_feats():
        feats_scr[1, pl.ds(i0, _BI), :] = f_new

    ew = jnp.sum(m * wv[0][None, :, :], axis=2)
    scale = inv_r * ew * (1.0 / _N)
    v0 = jnp.sum(d0 * scale, axis=1, keepdims=True)
    v1 = jnp.sum(d1 * scale, axis=1, keepdims=True)
    v2 = jnp.sum(d2 * scale, axis=1, keepdims=True)
    vcol = jnp.concatenate([v0, v1, v2], axis=1)

    @pl.when(l == 0)
    def _vec_init():
        vec_ref[pl.ds(i0, _BI), :] = vcol

    @pl.when(l > 0)
    def _vec_acc():
        vec_ref[pl.ds(i0, _BI), :] = vec_ref[pl.ds(i0, _BI), :] + vcol

    @pl.when(l == _L - 1)
    def _readout():
        t = f_new - jnp.max(f_new, axis=1, keepdims=True)
        e = jnp.exp(t)
        sm = e / jnp.sum(e, axis=1, keepdims=True)
        inv_ref[pl.ds(i0, _BI), :] = jax.lax.dot_general(
            sm, wout[:], (((1,), (0,)), ((), ())),
            preferred_element_type=jnp.float32) + bout[:]


def kernel(x, h, embed, Wr1, br1, Wr2, br2, Wm, bm, Wv, W_out, b_out):
    n, mult = x.shape[0], x.shape[1]
    pos = x[:, 0, :]
    pos_row = pos.T
    wr1 = Wr1
    br1_3 = br1.reshape(_L, 1, _D)
    br2_3 = br2.reshape(_L, 1, _D)
    bm_3 = bm.reshape(_L, 1, _D)
    wv_3 = Wv.reshape(_L, 1, _D)
    bout = b_out.reshape(1, _D)

    vec, inv = pl.pallas_call(
        _mace_body,
        grid=(_L, _NBLK),
        in_specs=[
            pl.BlockSpec((_N, 3), lambda l, ib: (0, 0)),
            pl.BlockSpec((3, _N), lambda l, ib: (0, 0)),
            pl.BlockSpec((1, _D), lambda l, ib: (0, 0)),
            pl.BlockSpec((1, 1, _D), lambda l, ib: (l, 0, 0)),
            pl.BlockSpec((1, 1, _D), lambda l, ib: (l, 0, 0)),
            pl.BlockSpec((1, _D, _D), lambda l, ib: (l, 0, 0)),
            pl.BlockSpec((1, 1, _D), lambda l, ib: (l, 0, 0)),
            pl.BlockSpec((1, _D, _D), lambda l, ib: (l, 0, 0)),
            pl.BlockSpec((1, 1, _D), lambda l, ib: (l, 0, 0)),
            pl.BlockSpec((1, 1, _D), lambda l, ib: (l, 0, 0)),
            pl.BlockSpec((_D, _D), lambda l, ib: (0, 0)),
            pl.BlockSpec((1, _D), lambda l, ib: (0, 0)),
        ],
        out_specs=[
            pl.BlockSpec((_N, 3), lambda l, ib: (0, 0)),
            pl.BlockSpec((_N, _D), lambda l, ib: (0, 0)),
        ],
        out_shape=[
            jax.ShapeDtypeStruct((_N, 3), jnp.float32),
            jax.ShapeDtypeStruct((_N, _D), jnp.float32),
        ],
        scratch_shapes=[pltpu.VMEM((_L, _N, _D), jnp.float32)],
        compiler_params=pltpu.CompilerParams(
            dimension_semantics=("arbitrary", "arbitrary")),
    )(pos, pos_row, embed, wr1, br1_3, Wr2, br2_3, Wm, bm_3, wv_3,
      W_out, bout)

    return vec.reshape(n, mult, 1, 3), inv.reshape(n, mult, _D)


# speedup vs baseline: 31.3168x; 1.3686x over previous
import jax
import jax.numpy as jnp
from jax.experimental import pallas as pl
from jax.experimental.pallas import tpu as pltpu

_N = 512
_D = 64
_L = 2
_BI = 32
_NBLK = _N // _BI
_RLEN_DIAG = 1e-3


def _silu(z):
    t = 0.5 * z
    return t * (1.0 + jnp.tanh(t))


def _mace_body(pos_col, pos_row, embed_r, embed_c, w1h, b1h, wr2, br2c,
               wm, bm_r, wvc, wout, bout_r, vec_ref, inv_ref,
               row_scr, t_scr):
    l = pl.program_id(0)
    ib = pl.program_id(1)
    i0 = ib * _BI

    pc = pos_col[pl.ds(i0, _BI), :]
    pr = pos_row[:]
    d0 = pc[:, 0:1] - pr[0:1, :]
    d1 = pc[:, 1:2] - pr[1:2, :]
    d2 = pc[:, 2:3] - pr[2:3, :]
    r2 = d0 * d0 + d1 * d1 + d2 * d2 + 1e-6
    rlen = jnp.sqrt(r2)
    inv_r = 1.0 / rlen

    feats_t = jnp.where(l == 0,
                        jnp.broadcast_to(embed_c[:], (_D, _N)),
                        t_scr[:])

    w1c = w1h[0][:, :, None]
    b1c = b1h[0][:, :, None]
    p = rlen[None, :, :] * w1c + b1c
    hid = p * (1.0 + jnp.tanh(p))
    radial = jax.lax.dot_general(
        wr2[0], hid, (((0,), (0,)), ((), ())),
        preferred_element_type=jnp.float32) + br2c[0][:, :, None]

    m = radial * feats_t[:, None, :]
    ew = jnp.sum(m * wvc[0][:, :, None], axis=0)

    agg_t = jnp.sum(m, axis=2)
    pd = _RLEN_DIAG * w1h[0] + b1h[0]
    hid_d = pd * (1.0 + jnp.tanh(pd))
    radial_d = jax.lax.dot_general(
        wr2[0], hid_d, (((0,), (0,)), ((), ())),
        preferred_element_type=jnp.float32) + br2c[0]

    f_blk = jnp.where(l == 0,
                      jnp.broadcast_to(embed_r[:], (_BI, _D)),
                      row_scr[pl.ds(i0, _BI), :])
    agg = (jnp.transpose(agg_t)
           - f_blk * jnp.transpose(radial_d)) * (1.0 / _N)
    z = jax.lax.dot_general(
        agg, wm[0], (((1,), (0,)), ((), ())),
        preferred_element_type=jnp.float32) + bm_r[0]
    f_new = f_blk + _silu(z)

    @pl.when(l == 0)
    def _store_feats():
        row_scr[pl.ds(i0, _BI), :] = f_new

    @pl.when(jnp.logical_and(l == 0, ib == _NBLK - 1))
    def _build_transposed():
        t_scr[:] = jnp.transpose(row_scr[:])

    scale = inv_r * ew * (1.0 / _N)
    v0 = jnp.sum(d0 * scale, axis=1, keepdims=True)
    v1 = jnp.sum(d1 * scale, axis=1, keepdims=True)
    v2 = jnp.sum(d2 * scale, axis=1, keepdims=True)
    vcol = jnp.concatenate([v0, v1, v2], axis=1)

    @pl.when(l == 0)
    def _vec_init():
        vec_ref[pl.ds(i0, _BI), :] = vcol

    @pl.when(l > 0)
    def _vec_acc():
        vec_ref[pl.ds(i0, _BI), :] = vec_ref[pl.ds(i0, _BI), :] + vcol

    @pl.when(l == _L - 1)
    def _readout():
        t = f_new - jnp.max(f_new, axis=1, keepdims=True)
        e = jnp.exp(t)
        sm = e / jnp.sum(e, axis=1, keepdims=True)
        inv_ref[pl.ds(i0, _BI), :] = jax.lax.dot_general(
            sm, wout[:], (((1,), (0,)), ((), ())),
            preferred_element_type=jnp.float32) + bout_r[:]


def kernel(x, h, embed, Wr1, br1, Wr2, br2, Wm, bm, Wv, W_out, b_out):
    n, mult = x.shape[0], x.shape[1]
    pos = x[:, 0, :]
    pos_row = pos.T
    w1h = Wr1.transpose(0, 2, 1) * 0.5
    b1h = br1.reshape(_L, _D, 1) * 0.5
    br2c = br2.reshape(_L, _D, 1)
    bm_r = bm.reshape(_L, 1, _D)
    embed_c = embed.T
    bout_r = b_out.reshape(1, _D)

    vec, inv = pl.pallas_call(
        _mace_body,
        grid=(_L, _NBLK),
        in_specs=[
            pl.BlockSpec((_N, 3), lambda l, ib: (0, 0)),
            pl.BlockSpec((3, _N), lambda l, ib: (0, 0)),
            pl.BlockSpec((1, _D), lambda l, ib: (0, 0)),
            pl.BlockSpec((_D, 1), lambda l, ib: (0, 0)),
            pl.BlockSpec((1, _D, 1), lambda l, ib: (l, 0, 0)),
            pl.BlockSpec((1, _D, 1), lambda l, ib: (l, 0, 0)),
            pl.BlockSpec((1, _D, _D), lambda l, ib: (l, 0, 0)),
            pl.BlockSpec((1, _D, 1), lambda l, ib: (l, 0, 0)),
            pl.BlockSpec((1, _D, _D), lambda l, ib: (l, 0, 0)),
            pl.BlockSpec((1, 1, _D), lambda l, ib: (l, 0, 0)),
            pl.BlockSpec((1, _D, 1), lambda l, ib: (l, 0, 0)),
            pl.BlockSpec((_D, _D), lambda l, ib: (0, 0)),
            pl.BlockSpec((1, _D), lambda l, ib: (0, 0)),
        ],
        out_specs=[
            pl.BlockSpec((_N, 3), lambda l, ib: (0, 0)),
            pl.BlockSpec((_N, _D), lambda l, ib: (0, 0)),
        ],
        out_shape=[
            jax.ShapeDtypeStruct((_N, 3), jnp.float32),
            jax.ShapeDtypeStruct((_N, _D), jnp.float32),
        ],
        scratch_shapes=[pltpu.VMEM((_N, _D), jnp.float32),
                        pltpu.VMEM((_D, _N), jnp.float32)],
        compiler_params=pltpu.CompilerParams(
            dimension_semantics=("arbitrary", "arbitrary")),
    )(pos, pos_row, embed, embed_c, w1h, b1h, Wr2, br2c, Wm, bm_r, Wv,
      W_out, bout_r)

    return vec.reshape(n, mult, 1, 3), inv.reshape(n, mult, _D)


# speedup vs baseline: 32.2629x; 1.0302x over previous
import jax
import jax.numpy as jnp
from jax.experimental import pallas as pl
from jax.experimental.pallas import tpu as pltpu

_N = 512
_D = 64
_L = 2
_BI = 32
_NBLK = _N // _BI
_RLEN_DIAG = 1e-3


def _silu(z):
    t = 0.5 * z
    return t * (1.0 + jnp.tanh(t))


def _mace_body(pos_col, pos_row, embed_r, embed_c, w1h, b1h, wr2, br2c,
               wm, bm_r, wvc, wout, bout_r, vec_ref, inv_ref,
               row_scr, t_scr):
    l = pl.program_id(0)
    ib = pl.program_id(1)
    i0 = ib * _BI

    pc = pos_col[pl.ds(i0, _BI), :]
    pr = pos_row[:]
    d0 = pc[:, 0:1] - pr[0:1, :]
    d1 = pc[:, 1:2] - pr[1:2, :]
    d2 = pc[:, 2:3] - pr[2:3, :]
    r2 = d0 * d0 + d1 * d1 + d2 * d2 + 1e-6
    rlen = jnp.sqrt(r2)
    inv_r = 1.0 / rlen

    feats_t = jnp.where(l == 0,
                        jnp.broadcast_to(embed_c[:], (_D, _N)),
                        t_scr[:])

    w1c = w1h[0][:, :, None]
    b1c = b1h[0][:, :, None]
    p = rlen[None, :, :] * w1c + b1c
    hid = p * (1.0 + jnp.tanh(p))
    radial = jax.lax.dot_general(
        wr2[0].astype(jnp.bfloat16), hid.astype(jnp.bfloat16),
        (((0,), (0,)), ((), ())),
        preferred_element_type=jnp.float32) + br2c[0][:, :, None]

    m = radial * feats_t[:, None, :]
    m16 = m.astype(jnp.bfloat16).astype(jnp.float32)
    wv16 = wvc[0].astype(jnp.bfloat16).astype(jnp.float32)
    ew = jnp.sum(m16 * wv16[:, :, None], axis=0)

    agg_t = jnp.sum(m, axis=2)
    pd = _RLEN_DIAG * w1h[0] + b1h[0]
    hid_d = pd * (1.0 + jnp.tanh(pd))
    radial_d = jax.lax.dot_general(
        wr2[0].astype(jnp.bfloat16), hid_d.astype(jnp.bfloat16),
        (((0,), (0,)), ((), ())),
        preferred_element_type=jnp.float32) + br2c[0]

    f_blk = jnp.where(l == 0,
                      jnp.broadcast_to(embed_r[:], (_BI, _D)),
                      row_scr[pl.ds(i0, _BI), :])
    agg = (jnp.transpose(agg_t)
           - f_blk * jnp.transpose(radial_d)) * (1.0 / _N)
    z = jax.lax.dot_general(
        agg, wm[0], (((1,), (0,)), ((), ())),
        preferred_element_type=jnp.float32) + bm_r[0]
    f_new = f_blk + _silu(z)

    @pl.when(l == 0)
    def _store_feats():
        row_scr[pl.ds(i0, _BI), :] = f_new

    @pl.when(jnp.logical_and(l == 0, ib == _NBLK - 1))
    def _build_transposed():
        t_scr[:] = jnp.transpose(row_scr[:])

    scale = inv_r * ew * (1.0 / _N)
    v0 = jnp.sum(d0 * scale, axis=1, keepdims=True)
    v1 = jnp.sum(d1 * scale, axis=1, keepdims=True)
    v2 = jnp.sum(d2 * scale, axis=1, keepdims=True)
    vcol = jnp.concatenate([v0, v1, v2], axis=1)

    @pl.when(l == 0)
    def _vec_init():
        vec_ref[pl.ds(i0, _BI), :] = vcol

    @pl.when(l > 0)
    def _vec_acc():
        vec_ref[pl.ds(i0, _BI), :] = vec_ref[pl.ds(i0, _BI), :] + vcol

    @pl.when(l == _L - 1)
    def _readout():
        t = f_new - jnp.max(f_new, axis=1, keepdims=True)
        e = jnp.exp(t)
        sm = e / jnp.sum(e, axis=1, keepdims=True)
        inv_ref[pl.ds(i0, _BI), :] = jax.lax.dot_general(
            sm, wout[:], (((1,), (0,)), ((), ())),
            preferred_element_type=jnp.float32) + bout_r[:]


def kernel(x, h, embed, Wr1, br1, Wr2, br2, Wm, bm, Wv, W_out, b_out):
    n, mult = x.shape[0], x.shape[1]
    pos = x[:, 0, :]
    pos_row = pos.T
    w1h = Wr1.transpose(0, 2, 1) * 0.5
    b1h = br1.reshape(_L, _D, 1) * 0.5
    br2c = br2.reshape(_L, _D, 1)
    bm_r = bm.reshape(_L, 1, _D)
    embed_c = embed.T
    bout_r = b_out.reshape(1, _D)

    vec, inv = pl.pallas_call(
        _mace_body,
        grid=(_L, _NBLK),
        in_specs=[
            pl.BlockSpec((_N, 3), lambda l, ib: (0, 0)),
            pl.BlockSpec((3, _N), lambda l, ib: (0, 0)),
            pl.BlockSpec((1, _D), lambda l, ib: (0, 0)),
            pl.BlockSpec((_D, 1), lambda l, ib: (0, 0)),
            pl.BlockSpec((1, _D, 1), lambda l, ib: (l, 0, 0)),
            pl.BlockSpec((1, _D, 1), lambda l, ib: (l, 0, 0)),
            pl.BlockSpec((1, _D, _D), lambda l, ib: (l, 0, 0)),
            pl.BlockSpec((1, _D, 1), lambda l, ib: (l, 0, 0)),
            pl.BlockSpec((1, _D, _D), lambda l, ib: (l, 0, 0)),
            pl.BlockSpec((1, 1, _D), lambda l, ib: (l, 0, 0)),
            pl.BlockSpec((1, _D, 1), lambda l, ib: (l, 0, 0)),
            pl.BlockSpec((_D, _D), lambda l, ib: (0, 0)),
            pl.BlockSpec((1, _D), lambda l, ib: (0, 0)),
        ],
        out_specs=[
            pl.BlockSpec((_N, 3), lambda l, ib: (0, 0)),
            pl.BlockSpec((_N, _D), lambda l, ib: (0, 0)),
        ],
        out_shape=[
            jax.ShapeDtypeStruct((_N, 3), jnp.float32),
            jax.ShapeDtypeStruct((_N, _D), jnp.float32),
        ],
        scratch_shapes=[pltpu.VMEM((_N, _D), jnp.float32),
                        pltpu.VMEM((_D, _N), jnp.float32)],
        compiler_params=pltpu.CompilerParams(
            dimension_semantics=("arbitrary", "arbitrary")),
    )(pos, pos_row, embed, embed_c, w1h, b1h, Wr2, br2c, Wm, bm_r, Wv,
      W_out, bout_r)

    return vec.reshape(n, mult, 1, 3), inv.reshape(n, mult, _D)


# speedup vs baseline: 36.7344x; 1.1386x over previous
import jax
import jax.numpy as jnp
from jax.experimental import pallas as pl
from jax.experimental.pallas import tpu as pltpu

_N = 512
_D = 64
_L = 2
_BI = 128
_NBLK = _N // _BI
_RLEN_DIAG = 1e-3


def _silu(z):
    t = 0.5 * z
    return t * (1.0 + jnp.tanh(t))


def _mace_body(pos_col, pos_row, embed_r, embed_c, w1h, b1h, wr2, br2c,
               wm, bm_r, wvc, wout, bout_r, vec_ref, inv_ref,
               row_scr, t_scr):
    l = pl.program_id(0)
    ib = pl.program_id(1)
    i0 = ib * _BI

    pc = pos_col[pl.ds(i0, _BI), :]
    pr = pos_row[:]
    d0 = pc[:, 0:1] - pr[0:1, :]
    d1 = pc[:, 1:2] - pr[1:2, :]
    d2 = pc[:, 2:3] - pr[2:3, :]
    r2 = d0 * d0 + d1 * d1 + d2 * d2 + 1e-6
    rlen = jnp.sqrt(r2)
    inv_r = 1.0 / rlen

    feats_t = jnp.where(l == 0,
                        jnp.broadcast_to(embed_c[:], (_D, _N)),
                        t_scr[:])

    w1c = w1h[0][:, :, None]
    b1c = b1h[0][:, :, None]
    p = rlen[None, :, :] * w1c + b1c
    hid = p * (1.0 + jnp.tanh(p))
    radial = jax.lax.dot_general(
        wr2[0].astype(jnp.bfloat16), hid.astype(jnp.bfloat16),
        (((0,), (0,)), ((), ())),
        preferred_element_type=jnp.float32) + br2c[0][:, :, None]

    m = radial * feats_t[:, None, :]
    m16 = m.astype(jnp.bfloat16).astype(jnp.float32)
    wv16 = wvc[0].astype(jnp.bfloat16).astype(jnp.float32)
    ew = jnp.sum(m16 * wv16[:, :, None], axis=0)

    agg_t = jnp.sum(m, axis=2)
    pd = _RLEN_DIAG * w1h[0] + b1h[0]
    hid_d = pd * (1.0 + jnp.tanh(pd))
    radial_d = jax.lax.dot_general(
        wr2[0].astype(jnp.bfloat16), hid_d.astype(jnp.bfloat16),
        (((0,), (0,)), ((), ())),
        preferred_element_type=jnp.float32) + br2c[0]

    f_blk = jnp.where(l == 0,
                      jnp.broadcast_to(embed_r[:], (_BI, _D)),
                      row_scr[pl.ds(i0, _BI), :])
    agg = (jnp.transpose(agg_t)
           - f_blk * jnp.transpose(radial_d)) * (1.0 / _N)
    z = jax.lax.dot_general(
        agg, wm[0], (((1,), (0,)), ((), ())),
        preferred_element_type=jnp.float32) + bm_r[0]
    f_new = f_blk + _silu(z)

    @pl.when(l == 0)
    def _store_feats():
        row_scr[pl.ds(i0, _BI), :] = f_new

    @pl.when(jnp.logical_and(l == 0, ib == _NBLK - 1))
    def _build_transposed():
        t_scr[:] = jnp.transpose(row_scr[:])

    scale = inv_r * ew * (1.0 / _N)
    v0 = jnp.sum(d0 * scale, axis=1, keepdims=True)
    v1 = jnp.sum(d1 * scale, axis=1, keepdims=True)
    v2 = jnp.sum(d2 * scale, axis=1, keepdims=True)
    vcol = jnp.concatenate([v0, v1, v2], axis=1)

    @pl.when(l == 0)
    def _vec_init():
        vec_ref[pl.ds(i0, _BI), :] = vcol

    @pl.when(l > 0)
    def _vec_acc():
        vec_ref[pl.ds(i0, _BI), :] = vec_ref[pl.ds(i0, _BI), :] + vcol

    @pl.when(l == _L - 1)
    def _readout():
        t = f_new - jnp.max(f_new, axis=1, keepdims=True)
        e = jnp.exp(t)
        sm = e / jnp.sum(e, axis=1, keepdims=True)
        inv_ref[pl.ds(i0, _BI), :] = jax.lax.dot_general(
            sm, wout[:], (((1,), (0,)), ((), ())),
            preferred_element_type=jnp.float32) + bout_r[:]


def kernel(x, h, embed, Wr1, br1, Wr2, br2, Wm, bm, Wv, W_out, b_out):
    n, mult = x.shape[0], x.shape[1]
    pos = x[:, 0, :]
    pos_row = pos.T
    w1h = Wr1.transpose(0, 2, 1) * 0.5
    b1h = br1.reshape(_L, _D, 1) * 0.5
    br2c = br2.reshape(_L, _D, 1)
    bm_r = bm.reshape(_L, 1, _D)
    embed_c = embed.T
    bout_r = b_out.reshape(1, _D)

    vec, inv = pl.pallas_call(
        _mace_body,
        grid=(_L, _NBLK),
        in_specs=[
            pl.BlockSpec((_N, 3), lambda l, ib: (0, 0)),
            pl.BlockSpec((3, _N), lambda l, ib: (0, 0)),
            pl.BlockSpec((1, _D), lambda l, ib: (0, 0)),
            pl.BlockSpec((_D, 1), lambda l, ib: (0, 0)),
            pl.BlockSpec((1, _D, 1), lambda l, ib: (l, 0, 0)),
            pl.BlockSpec((1, _D, 1), lambda l, ib: (l, 0, 0)),
            pl.BlockSpec((1, _D, _D), lambda l, ib: (l, 0, 0)),
            pl.BlockSpec((1, _D, 1), lambda l, ib: (l, 0, 0)),
            pl.BlockSpec((1, _D, _D), lambda l, ib: (l, 0, 0)),
            pl.BlockSpec((1, 1, _D), lambda l, ib: (l, 0, 0)),
            pl.BlockSpec((1, _D, 1), lambda l, ib: (l, 0, 0)),
            pl.BlockSpec((_D, _D), lambda l, ib: (0, 0)),
            pl.BlockSpec((1, _D), lambda l, ib: (0, 0)),
        ],
        out_specs=[
            pl.BlockSpec((_N, 3), lambda l, ib: (0, 0)),
            pl.BlockSpec((_N, _D), lambda l, ib: (0, 0)),
        ],
        out_shape=[
            jax.ShapeDtypeStruct((_N, 3), jnp.float32),
            jax.ShapeDtypeStruct((_N, _D), jnp.float32),
        ],
        scratch_shapes=[pltpu.VMEM((_N, _D), jnp.float32),
                        pltpu.VMEM((_D, _N), jnp.float32)],
        compiler_params=pltpu.CompilerParams(
            dimension_semantics=("arbitrary", "arbitrary")),
    )(pos, pos_row, embed, embed_c, w1h, b1h, Wr2, br2c, Wm, bm_r, Wv,
      W_out, bout_r)

    return vec.reshape(n, mult, 1, 3), inv.reshape(n, mult, _D)


# speedup vs baseline: 39.0347x; 1.0626x over previous
import jax
import jax.numpy as jnp
from jax.experimental import pallas as pl
from jax.experimental.pallas import tpu as pltpu

_N = 512
_D = 64
_L = 2
_T = 128
_NT = _N // _T
_NPAIR = (_NT * (_NT + 1)) // 2
_RLEN_DIAG = 1e-3


def _silu(z):
    t = 0.5 * z
    return t * (1.0 + jnp.tanh(t))


def _bf(v):
    return v.astype(jnp.bfloat16)


def _mace_body(pos_col, pos_row, embed_c, w1h, b1h, wr2, br2c, wm, bmc,
               wvc, wout, boutc, vec_ref, invt_ref, t_scr, agg_scr):
    l = pl.program_id(0)
    p = pl.program_id(1)

    @pl.when(jnp.logical_and(l == 0, p == 0))
    def _init():
        t_scr[:] = jnp.broadcast_to(embed_c[:], (_D, _N))
        vec_ref[:] = jnp.zeros((_N, 3), jnp.float32)

    @pl.when(p == 0)
    def _zero_agg():
        agg_scr[:] = jnp.zeros((_D, _N), jnp.float32)

    wv16 = wvc[0].astype(jnp.bfloat16).astype(jnp.float32)

    def _halfside(radial, ra, sb):
        i0 = ra * _T
        j0 = sb * _T
        pc = pos_col[pl.ds(i0, _T), :]
        pr = pos_row[:, pl.ds(j0, _T)]
        d0 = pc[:, 0:1] - pr[0:1, :]
        d1 = pc[:, 1:2] - pr[1:2, :]
        d2 = pc[:, 2:3] - pr[2:3, :]
        r2 = d0 * d0 + d1 * d1 + d2 * d2 + 1e-6
        inv_r = 1.0 / jnp.sqrt(r2)
        feats_sb = t_scr[:, pl.ds(j0, _T)]
        m = radial * feats_sb[:, None, :]
        m16 = m.astype(jnp.bfloat16).astype(jnp.float32)
        ew = jnp.sum(m16 * wv16[:, :, None], axis=0)
        agg_scr[:, pl.ds(i0, _T)] = (agg_scr[:, pl.ds(i0, _T)]
                                     + jnp.sum(m, axis=2))
        scale = inv_r * ew * (1.0 / _N)
        v0 = jnp.sum(d0 * scale, axis=1, keepdims=True)
        v1 = jnp.sum(d1 * scale, axis=1, keepdims=True)
        v2 = jnp.sum(d2 * scale, axis=1, keepdims=True)
        vcol = jnp.concatenate([v0, v1, v2], axis=1)
        vec_ref[pl.ds(i0, _T), :] = vec_ref[pl.ds(i0, _T), :] + vcol

    @pl.when(p < _NPAIR)
    def _pair_step():
        a = jnp.where(p < 4, 0, jnp.where(p < 7, 1, jnp.where(p < 9, 2, 3)))
        b = jnp.where(p < 4, p, jnp.where(p < 7, p - 3,
                                          jnp.where(p < 9, p - 5, p - 6)))
        i0 = a * _T
        j0 = b * _T
        pc = pos_col[pl.ds(i0, _T), :]
        pr = pos_row[:, pl.ds(j0, _T)]
        d0 = pc[:, 0:1] - pr[0:1, :]
        d1 = pc[:, 1:2] - pr[1:2, :]
        d2 = pc[:, 2:3] - pr[2:3, :]
        rlen = jnp.sqrt(d0 * d0 + d1 * d1 + d2 * d2 + 1e-6)
        w1c = w1h[0][:, :, None]
        b1c = b1h[0][:, :, None]
        q = rlen[None, :, :] * w1c + b1c
        hid = q * (1.0 + jnp.tanh(q))
        radial = jax.lax.dot_general(
            _bf(wr2[0]), _bf(hid), (((0,), (0,)), ((), ())),
            preferred_element_type=jnp.float32) + br2c[0][:, :, None]
        _halfside(radial, a, b)

        @pl.when(a != b)
        def _reverse():
            _halfside(jnp.transpose(radial, (0, 2, 1)), b, a)

    @pl.when(p == _NPAIR)
    def _update_step():
        f_old = t_scr[:]
        pd = _RLEN_DIAG * w1h[0] + b1h[0]
        hid_d = pd * (1.0 + jnp.tanh(pd))
        radial_d = jax.lax.dot_general(
            _bf(wr2[0]), _bf(hid_d), (((0,), (0,)), ((), ())),
            preferred_element_type=jnp.float32) + br2c[0]
        agg = (agg_scr[:] - f_old * radial_d) * (1.0 / _N)
        z = jax.lax.dot_general(
            _bf(wm[0]), _bf(agg), (((0,), (0,)), ((), ())),
            preferred_element_type=jnp.float32) + bmc[0]
        f_new = f_old + _silu(z)
        t_scr[:] = f_new

        @pl.when(l == _L - 1)
        def _readout():
            t = f_new - jnp.max(f_new, axis=0, keepdims=True)
            e = jnp.exp(t)
            sm = e / jnp.sum(e, axis=0, keepdims=True)
            invt_ref[:] = jax.lax.dot_general(
                _bf(wout[:]), _bf(sm), (((0,), (0,)), ((), ())),
                preferred_element_type=jnp.float32) + boutc[:]


def kernel(x, h, embed, Wr1, br1, Wr2, br2, Wm, bm, Wv, W_out, b_out):
    n, mult = x.shape[0], x.shape[1]
    pos = x[:, 0, :]
    pos_row = pos.T
    w1h = Wr1.transpose(0, 2, 1) * 0.5
    b1h = br1.reshape(_L, _D, 1) * 0.5
    br2c = br2.reshape(_L, _D, 1)
    bmc = bm.reshape(_L, _D, 1)
    embed_c = embed.T
    boutc = b_out.reshape(_D, 1)

    vec, invt = pl.pallas_call(
        _mace_body,
        grid=(_L, _NPAIR + 1),
        in_specs=[
            pl.BlockSpec((_N, 3), lambda l, p: (0, 0)),
            pl.BlockSpec((3, _N), lambda l, p: (0, 0)),
            pl.BlockSpec((_D, 1), lambda l, p: (0, 0)),
            pl.BlockSpec((1, _D, 1), lambda l, p: (l, 0, 0)),
            pl.BlockSpec((1, _D, 1), lambda l, p: (l, 0, 0)),
            pl.BlockSpec((1, _D, _D), lambda l, p: (l, 0, 0)),
            pl.BlockSpec((1, _D, 1), lambda l, p: (l, 0, 0)),
            pl.BlockSpec((1, _D, _D), lambda l, p: (l, 0, 0)),
            pl.BlockSpec((1, _D, 1), lambda l, p: (l, 0, 0)),
            pl.BlockSpec((1, _D, 1), lambda l, p: (l, 0, 0)),
            pl.BlockSpec((_D, _D), lambda l, p: (0, 0)),
            pl.BlockSpec((_D, 1), lambda l, p: (0, 0)),
        ],
        out_specs=[
            pl.BlockSpec((_N, 3), lambda l, p: (0, 0)),
            pl.BlockSpec((_D, _N), lambda l, p: (0, 0)),
        ],
        out_shape=[
            jax.ShapeDtypeStruct((_N, 3), jnp.float32),
            jax.ShapeDtypeStruct((_D, _N), jnp.float32),
        ],
        scratch_shapes=[pltpu.VMEM((_D, _N), jnp.float32),
                        pltpu.VMEM((_D, _N), jnp.float32)],
        compiler_params=pltpu.CompilerParams(
            dimension_semantics=("arbitrary", "arbitrary")),
    )(pos, pos_row, embed_c, w1h, b1h, Wr2, br2c, Wm, bmc, Wv,
      W_out, boutc)

    return vec.reshape(n, mult, 1, 3), invt.T.reshape(n, mult, _D)


# speedup vs baseline: 39.5771x; 1.0139x over previous
import jax
import jax.numpy as jnp
from jax.experimental import pallas as pl
from jax.experimental.pallas import tpu as pltpu

_N = 512
_D = 64
_L = 2
_T = 128
_NT = _N // _T
_NPAIR = (_NT * (_NT + 1)) // 2
_RLEN_DIAG = 1e-3


def _silu(z):
    t = 0.5 * z
    return t * (1.0 + jnp.tanh(t))


def _bf(v):
    return v.astype(jnp.bfloat16)


def _mace_body(pos_col, pos_row, embed_c, w1h, b1h, wr2, br2c, wm, bmc,
               wvc, wout, boutc, vec_ref, invt_ref, t_scr, agg_scr):
    l = pl.program_id(0)
    p = pl.program_id(1)

    @pl.when(jnp.logical_and(l == 0, p == 0))
    def _init():
        t_scr[:] = jnp.broadcast_to(embed_c[:], (_D, _N))
        vec_ref[:] = jnp.zeros((_N, 3), jnp.float32)

    @pl.when(p == 0)
    def _zero_agg():
        agg_scr[:] = jnp.zeros((_D, _N), jnp.float32)

    wv16 = wvc[0].astype(jnp.bfloat16).astype(jnp.float32)

    def _halfside(radial, ra, sb):
        i0 = ra * _T
        j0 = sb * _T
        pc = pos_col[pl.ds(i0, _T), :]
        pr = pos_row[:, pl.ds(j0, _T)]
        d0 = pc[:, 0:1] - pr[0:1, :]
        d1 = pc[:, 1:2] - pr[1:2, :]
        d2 = pc[:, 2:3] - pr[2:3, :]
        r2 = d0 * d0 + d1 * d1 + d2 * d2 + 1e-6
        inv_r = 1.0 / jnp.sqrt(r2)
        feats_sb = t_scr[:, pl.ds(j0, _T)]
        m = radial * feats_sb[:, None, :]
        m16 = m.astype(jnp.bfloat16).astype(jnp.float32)
        ew = jnp.sum(m16 * wv16[:, :, None], axis=0)
        agg_scr[:, pl.ds(i0, _T)] = (agg_scr[:, pl.ds(i0, _T)]
                                     + jnp.sum(m, axis=2))
        scale = inv_r * ew * (1.0 / _N)
        v0 = jnp.sum(d0 * scale, axis=1, keepdims=True)
        v1 = jnp.sum(d1 * scale, axis=1, keepdims=True)
        v2 = jnp.sum(d2 * scale, axis=1, keepdims=True)
        vcol = jnp.concatenate([v0, v1, v2], axis=1)
        vec_ref[pl.ds(i0, _T), :] = vec_ref[pl.ds(i0, _T), :] + vcol

    def _do_pair(pp):
        a = jnp.where(pp < 4, 0, jnp.where(pp < 7, 1, jnp.where(pp < 9, 2, 3)))
        b = jnp.where(pp < 4, pp, jnp.where(pp < 7, pp - 3,
                                            jnp.where(pp < 9, pp - 5, pp - 6)))
        i0 = a * _T
        j0 = b * _T
        pc = pos_col[pl.ds(i0, _T), :]
        pr = pos_row[:, pl.ds(j0, _T)]
        d0 = pc[:, 0:1] - pr[0:1, :]
        d1 = pc[:, 1:2] - pr[1:2, :]
        d2 = pc[:, 2:3] - pr[2:3, :]
        rlen = jnp.sqrt(d0 * d0 + d1 * d1 + d2 * d2 + 1e-6)
        w1c = w1h[0][:, :, None]
        b1c = b1h[0][:, :, None]
        q = rlen[None, :, :] * w1c + b1c
        hid = q * (1.0 + jnp.tanh(q))
        radial = jax.lax.dot_general(
            _bf(wr2[0]), _bf(hid), (((0,), (0,)), ((), ())),
            preferred_element_type=jnp.float32) + br2c[0][:, :, None]
        _halfside(radial, a, b)

        @pl.when(a != b)
        def _reverse():
            _halfside(jnp.transpose(radial, (0, 2, 1)), b, a)

    @pl.when(p < _NPAIR // 2)
    def _pair_step():
        _do_pair(2 * p)
        _do_pair(2 * p + 1)

    @pl.when(p == _NPAIR // 2)
    def _update_step():
        f_old = t_scr[:]
        pd = _RLEN_DIAG * w1h[0] + b1h[0]
        hid_d = pd * (1.0 + jnp.tanh(pd))
        radial_d = jax.lax.dot_general(
            _bf(wr2[0]), _bf(hid_d), (((0,), (0,)), ((), ())),
            preferred_element_type=jnp.float32) + br2c[0]
        agg = (agg_scr[:] - f_old * radial_d) * (1.0 / _N)
        z = jax.lax.dot_general(
            _bf(wm[0]), _bf(agg), (((0,), (0,)), ((), ())),
            preferred_element_type=jnp.float32) + bmc[0]
        f_new = f_old + _silu(z)
        t_scr[:] = f_new

        @pl.when(l == _L - 1)
        def _readout():
            t = f_new - jnp.max(f_new, axis=0, keepdims=True)
            e = jnp.exp(t)
            sm = e / jnp.sum(e, axis=0, keepdims=True)
            invt_ref[:] = jax.lax.dot_general(
                _bf(wout[:]), _bf(sm), (((0,), (0,)), ((), ())),
                preferred_element_type=jnp.float32) + boutc[:]


def kernel(x, h, embed, Wr1, br1, Wr2, br2, Wm, bm, Wv, W_out, b_out):
    n, mult = x.shape[0], x.shape[1]
    pos = x[:, 0, :]
    pos_row = pos.T
    w1h = Wr1.transpose(0, 2, 1) * 0.5
    b1h = br1.reshape(_L, _D, 1) * 0.5
    br2c = br2.reshape(_L, _D, 1)
    bmc = bm.reshape(_L, _D, 1)
    embed_c = embed.T
    boutc = b_out.reshape(_D, 1)

    vec, invt = pl.pallas_call(
        _mace_body,
        grid=(_L, _NPAIR // 2 + 1),
        in_specs=[
            pl.BlockSpec((_N, 3), lambda l, p: (0, 0)),
            pl.BlockSpec((3, _N), lambda l, p: (0, 0)),
            pl.BlockSpec((_D, 1), lambda l, p: (0, 0)),
            pl.BlockSpec((1, _D, 1), lambda l, p: (l, 0, 0)),
            pl.BlockSpec((1, _D, 1), lambda l, p: (l, 0, 0)),
            pl.BlockSpec((1, _D, _D), lambda l, p: (l, 0, 0)),
            pl.BlockSpec((1, _D, 1), lambda l, p: (l, 0, 0)),
            pl.BlockSpec((1, _D, _D), lambda l, p: (l, 0, 0)),
            pl.BlockSpec((1, _D, 1), lambda l, p: (l, 0, 0)),
            pl.BlockSpec((1, _D, 1), lambda l, p: (l, 0, 0)),
            pl.BlockSpec((_D, _D), lambda l, p: (0, 0)),
            pl.BlockSpec((_D, 1), lambda l, p: (0, 0)),
        ],
        out_specs=[
            pl.BlockSpec((_N, 3), lambda l, p: (0, 0)),
            pl.BlockSpec((_D, _N), lambda l, p: (0, 0)),
        ],
        out_shape=[
            jax.ShapeDtypeStruct((_N, 3), jnp.float32),
            jax.ShapeDtypeStruct((_D, _N), jnp.float32),
        ],
        scratch_shapes=[pltpu.VMEM((_D, _N), jnp.float32),
                        pltpu.VMEM((_D, _N), jnp.float32)],
        compiler_params=pltpu.CompilerParams(
            dimension_semantics=("arbitrary", "arbitrary")),
    )(pos, pos_row, embed_c, w1h, b1h, Wr2, br2c, Wm, bmc, Wv,
      W_out, boutc)

    return vec.reshape(n, mult, 1, 3), invt.T.reshape(n, mult, _D)


# speedup vs baseline: 39.9798x; 1.0102x over previous
import jax
import jax.numpy as jnp
from jax.experimental import pallas as pl
from jax.experimental.pallas import tpu as pltpu

_N = 512
_D = 64
_L = 2
_T = 128
_NT = _N // _T
_NPAIR = (_NT * (_NT + 1)) // 2
_PPS = 5
_RLEN_DIAG = 1e-3


def _silu(z):
    t = 0.5 * z
    return t * (1.0 + jnp.tanh(t))


def _bf(v):
    return v.astype(jnp.bfloat16)


def _mace_body(pos_col, pos_row, embed_c, w1h, b1h, wr2, br2c, wm, bmc,
               wvc, wout, boutc, vec_ref, invt_ref, t_scr, agg_scr):
    l = pl.program_id(0)
    p = pl.program_id(1)

    @pl.when(jnp.logical_and(l == 0, p == 0))
    def _init():
        t_scr[:] = jnp.broadcast_to(embed_c[:], (_D, _N))
        vec_ref[:] = jnp.zeros((_N, 3), jnp.float32)

    @pl.when(p == 0)
    def _zero_agg():
        agg_scr[:] = jnp.zeros((_D, _N), jnp.float32)

    wv16 = wvc[0].astype(jnp.bfloat16).astype(jnp.float32)

    def _halfside(radial, ra, sb):
        i0 = ra * _T
        j0 = sb * _T
        pc = pos_col[pl.ds(i0, _T), :]
        pr = pos_row[:, pl.ds(j0, _T)]
        d0 = pc[:, 0:1] - pr[0:1, :]
        d1 = pc[:, 1:2] - pr[1:2, :]
        d2 = pc[:, 2:3] - pr[2:3, :]
        r2 = d0 * d0 + d1 * d1 + d2 * d2 + 1e-6
        inv_r = 1.0 / jnp.sqrt(r2)
        feats_sb = t_scr[:, pl.ds(j0, _T)]
        m = radial * feats_sb[:, None, :]
        m16 = m.astype(jnp.bfloat16).astype(jnp.float32)
        ew = jnp.sum(m16 * wv16[:, :, None], axis=0)
        agg_scr[:, pl.ds(i0, _T)] = (agg_scr[:, pl.ds(i0, _T)]
                                     + jnp.sum(m, axis=2))
        scale = inv_r * ew * (1.0 / _N)
        v0 = jnp.sum(d0 * scale, axis=1, keepdims=True)
        v1 = jnp.sum(d1 * scale, axis=1, keepdims=True)
        v2 = jnp.sum(d2 * scale, axis=1, keepdims=True)
        vcol = jnp.concatenate([v0, v1, v2], axis=1)
        vec_ref[pl.ds(i0, _T), :] = vec_ref[pl.ds(i0, _T), :] + vcol

    def _do_pair(pp):
        a = jnp.where(pp < 4, 0, jnp.where(pp < 7, 1, jnp.where(pp < 9, 2, 3)))
        b = jnp.where(pp < 4, pp, jnp.where(pp < 7, pp - 3,
                                            jnp.where(pp < 9, pp - 5, pp - 6)))
        i0 = a * _T
        j0 = b * _T
        pc = pos_col[pl.ds(i0, _T), :]
        pr = pos_row[:, pl.ds(j0, _T)]
        d0 = pc[:, 0:1] - pr[0:1, :]
        d1 = pc[:, 1:2] - pr[1:2, :]
        d2 = pc[:, 2:3] - pr[2:3, :]
        rlen = jnp.sqrt(d0 * d0 + d1 * d1 + d2 * d2 + 1e-6)
        w1c = w1h[0][:, :, None]
        b1c = b1h[0][:, :, None]
        q = rlen[None, :, :] * w1c + b1c
        hid = q * (1.0 + jnp.tanh(q))
        radial = jax.lax.dot_general(
            _bf(wr2[0]), _bf(hid), (((0,), (0,)), ((), ())),
            preferred_element_type=jnp.float32) + br2c[0][:, :, None]
        _halfside(radial, a, b)

        @pl.when(a != b)
        def _reverse():
            _halfside(jnp.transpose(radial, (0, 2, 1)), b, a)

    @pl.when(p < _NPAIR // _PPS)
    def _pair_step():
        for t in range(_PPS):
            _do_pair(_PPS * p + t)

    @pl.when(p == _NPAIR // _PPS)
    def _update_step():
        f_old = t_scr[:]
        pd = _RLEN_DIAG * w1h[0] + b1h[0]
        hid_d = pd * (1.0 + jnp.tanh(pd))
        radial_d = jax.lax.dot_general(
            _bf(wr2[0]), _bf(hid_d), (((0,), (0,)), ((), ())),
            preferred_element_type=jnp.float32) + br2c[0]
        agg = (agg_scr[:] - f_old * radial_d) * (1.0 / _N)
        z = jax.lax.dot_general(
            _bf(wm[0]), _bf(agg), (((0,), (0,)), ((), ())),
            preferred_element_type=jnp.float32) + bmc[0]
        f_new = f_old + _silu(z)
        t_scr[:] = f_new

        @pl.when(l == _L - 1)
        def _readout():
            t = f_new - jnp.max(f_new, axis=0, keepdims=True)
            e = jnp.exp(t)
            sm = e / jnp.sum(e, axis=0, keepdims=True)
            invt_ref[:] = jax.lax.dot_general(
                _bf(wout[:]), _bf(sm), (((0,), (0,)), ((), ())),
                preferred_element_type=jnp.float32) + boutc[:]


def kernel(x, h, embed, Wr1, br1, Wr2, br2, Wm, bm, Wv, W_out, b_out):
    n, mult = x.shape[0], x.shape[1]
    pos = x[:, 0, :]
    pos_row = pos.T
    w1h = Wr1.transpose(0, 2, 1) * 0.5
    b1h = br1.reshape(_L, _D, 1) * 0.5
    br2c = br2.reshape(_L, _D, 1)
    bmc = bm.reshape(_L, _D, 1)
    embed_c = embed.T
    boutc = b_out.reshape(_D, 1)

    vec, invt = pl.pallas_call(
        _mace_body,
        grid=(_L, _NPAIR // _PPS + 1),
        in_specs=[
            pl.BlockSpec((_N, 3), lambda l, p: (0, 0)),
            pl.BlockSpec((3, _N), lambda l, p: (0, 0)),
            pl.BlockSpec((_D, 1), lambda l, p: (0, 0)),
            pl.BlockSpec((1, _D, 1), lambda l, p: (l, 0, 0)),
            pl.BlockSpec((1, _D, 1), lambda l, p: (l, 0, 0)),
            pl.BlockSpec((1, _D, _D), lambda l, p: (l, 0, 0)),
            pl.BlockSpec((1, _D, 1), lambda l, p: (l, 0, 0)),
            pl.BlockSpec((1, _D, _D), lambda l, p: (l, 0, 0)),
            pl.BlockSpec((1, _D, 1), lambda l, p: (l, 0, 0)),
            pl.BlockSpec((1, _D, 1), lambda l, p: (l, 0, 0)),
            pl.BlockSpec((_D, _D), lambda l, p: (0, 0)),
            pl.BlockSpec((_D, 1), lambda l, p: (0, 0)),
        ],
        out_specs=[
            pl.BlockSpec((_N, 3), lambda l, p: (0, 0)),
            pl.BlockSpec((_D, _N), lambda l, p: (0, 0)),
        ],
        out_shape=[
            jax.ShapeDtypeStruct((_N, 3), jnp.float32),
            jax.ShapeDtypeStruct((_D, _N), jnp.float32),
        ],
        scratch_shapes=[pltpu.VMEM((_D, _N), jnp.float32),
                        pltpu.VMEM((_D, _N), jnp.float32)],
        compiler_params=pltpu.CompilerParams(
            dimension_semantics=("arbitrary", "arbitrary")),
    )(pos, pos_row, embed_c, w1h, b1h, Wr2, br2c, Wm, bmc, Wv,
      W_out, boutc)

    return vec.reshape(n, mult, 1, 3), invt.T.reshape(n, mult, _D)
